# Initial kernel scaffold; baseline (speedup 1.0000x reference)
import jax
import jax.numpy as jnp
from jax.experimental import pallas as pl


def kernel(vector_embeddings, scalar_embeddings, edge_vectors, edge_index, Wf, bf, w1, b1, w2, b2):
    raise NotImplementedError("write your pallas kernel here")


# speedup vs baseline: 12.9282x; 12.9282x over previous
import functools

import jax
import jax.numpy as jnp
from jax import lax
from jax.experimental import pallas as pl
from jax.experimental.pallas import tpu as pltpu
from jax.experimental.pallas import tpu_sc as plsc

CUTOFF = 5.0
NBASIS = 20


def _mlp_body(s_ref, w1_ref, b1_ref, w2_ref, b2_ref, o_ref):
    x = s_ref[...]
    h = jnp.dot(x, w1_ref[...], preferred_element_type=jnp.float32,
                precision=lax.Precision.HIGHEST) + b1_ref[...]
    h = h * jax.nn.sigmoid(h)
    o_ref[...] = jnp.dot(h, w2_ref[...], preferred_element_type=jnp.float32,
                         precision=lax.Precision.HIGHEST) + b2_ref[...]


def _payload_body(ev_ref, gnf_ref, gv0_ref, gv1_ref, gv2_ref, wf_ref, bf_ref,
                  pa_ref, p0_ref, p1_ref, p2_ref):
    ev = ev_ref[...]
    e0 = ev[:, 0:1]
    e1 = ev[:, 1:2]
    e2 = ev[:, 2:3]
    d = jnp.sqrt(e0 * e0 + e1 * e1 + e2 * e2)
    inv_d = 1.0 / d

    B = ev.shape[0]
    n = lax.broadcasted_iota(jnp.int32, (B, 128), 1).astype(jnp.float32) + 1.0
    bes = jnp.sqrt(2.0 / CUTOFF) * jnp.sin(n * (jnp.pi / CUTOFF) * d) * inv_d
    filt = jnp.dot(bes, wf_ref[...], preferred_element_type=jnp.float32,
                   precision=lax.Precision.HIGHEST) + bf_ref[...]

    u = d * (1.0 / CUTOFF)
    u2 = u * u
    u3 = u2 * u
    u6 = u3 * u3
    u7 = u6 * u
    u8 = u7 * u
    env = 1.0 - 28.0 * u6 + 48.0 * u7 - 21.0 * u8
    env = jnp.where(d < CUTOFF, env, 0.0)

    x = gnf_ref[...] * (filt * env)
    D = x.shape[1] // 3
    a = x[:, :D]
    b = x[:, D:2 * D]
    c = x[:, 2 * D:]
    pa_ref[...] = a
    p0_ref[...] = b * (e0 * inv_d) + c * gv0_ref[...]
    p1_ref[...] = b * (e1 * inv_d) + c * gv1_ref[...]
    p2_ref[...] = b * (e2 * inv_d) + c * gv2_ref[...]


def _mlp(scalar_embeddings, w1, b1, w2, b2):
    N, D = scalar_embeddings.shape
    TD = w2.shape[1]
    BN = 2000
    grid = (N // BN,)
    return pl.pallas_call(
        _mlp_body,
        grid=grid,
        in_specs=[
            pl.BlockSpec((BN, D), lambda i: (i, 0)),
            pl.BlockSpec((D, D), lambda i: (0, 0)),
            pl.BlockSpec((1, D), lambda i: (0, 0)),
            pl.BlockSpec((D, TD), lambda i: (0, 0)),
            pl.BlockSpec((1, TD), lambda i: (0, 0)),
        ],
        out_specs=pl.BlockSpec((BN, TD), lambda i: (i, 0)),
        out_shape=jax.ShapeDtypeStruct((N, TD), jnp.float32),
    )(scalar_embeddings, w1, b1.reshape(1, D), w2, b2.reshape(1, TD))


def _payloads(edge_vectors, gnf, gv0, gv1, gv2, wf_pad, bf):
    E, TD = gnf.shape
    D = TD // 3
    BE = 1600
    grid = (E // BE,)
    outs = pl.pallas_call(
        _payload_body,
        grid=grid,
        in_specs=[
            pl.BlockSpec((BE, 3), lambda i: (i, 0)),
            pl.BlockSpec((BE, TD), lambda i: (i, 0)),
            pl.BlockSpec((BE, D), lambda i: (i, 0)),
            pl.BlockSpec((BE, D), lambda i: (i, 0)),
            pl.BlockSpec((BE, D), lambda i: (i, 0)),
            pl.BlockSpec((128, TD), lambda i: (0, 0)),
            pl.BlockSpec((1, TD), lambda i: (0, 0)),
        ],
        out_specs=[pl.BlockSpec((BE, D), lambda i: (i, 0))] * 4,
        out_shape=[jax.ShapeDtypeStruct((E, D), jnp.float32)] * 4,
    )(edge_vectors, gnf, gv0, gv1, gv2, wf_pad, bf.reshape(1, TD))
    return outs


def _sc_gather(nf, vt0, vt1, vt2, neigh):
    N, TD = nf.shape
    D = vt0.shape[1]
    E = neigh.shape[0]
    BLK = 128
    nblk = E // BLK
    NW = 32
    per = (nblk + NW - 1) // NW
    mesh = plsc.VectorSubcoreMesh(core_axis_name="c", subcore_axis_name="s")

    @functools.partial(
        pl.kernel,
        out_type=(jax.ShapeDtypeStruct((E, TD), jnp.float32),
                  jax.ShapeDtypeStruct((E, D), jnp.float32),
                  jax.ShapeDtypeStruct((E, D), jnp.float32),
                  jax.ShapeDtypeStruct((E, D), jnp.float32)),
        mesh=mesh,
        scratch_types=[
            pltpu.VMEM((BLK,), jnp.int32),
            pltpu.VMEM((BLK, TD), jnp.float32),
            pltpu.VMEM((BLK, D), jnp.float32),
            pltpu.VMEM((BLK, D), jnp.float32),
            pltpu.VMEM((BLK, D), jnp.float32),
            pltpu.SemaphoreType.DMA,
            pltpu.SemaphoreType.DMA,
        ],
    )
    def k(nf_hbm, v0_hbm, v1_hbm, v2_hbm, idx_hbm,
          gnf_hbm, g0_hbm, g1_hbm, g2_hbm,
          idx_v, nf_v, b0_v, b1_v, b2_v, gsem, wsem):
        wid = lax.axis_index("s") * 2 + lax.axis_index("c")

        @pl.loop(0, per)
        def _(i):
            blk = wid + i * NW

            @pl.when(blk < nblk)
            def _():
                base = blk * BLK
                pltpu.sync_copy(idx_hbm.at[pl.ds(base, BLK)], idx_v)
                c0 = pltpu.async_copy(nf_hbm.at[idx_v], nf_v, gsem)
                c1 = pltpu.async_copy(v0_hbm.at[idx_v], b0_v, gsem)
                c2 = pltpu.async_copy(v1_hbm.at[idx_v], b1_v, gsem)
                c3 = pltpu.async_copy(v2_hbm.at[idx_v], b2_v, gsem)
                c0.wait()
                c1.wait()
                c2.wait()
                c3.wait()
                w0 = pltpu.async_copy(nf_v, gnf_hbm.at[pl.ds(base, BLK)], wsem)
                w1_ = pltpu.async_copy(b0_v, g0_hbm.at[pl.ds(base, BLK)], wsem)
                w2_ = pltpu.async_copy(b1_v, g1_hbm.at[pl.ds(base, BLK)], wsem)
                w3 = pltpu.async_copy(b2_v, g2_hbm.at[pl.ds(base, BLK)], wsem)
                w0.wait()
                w1_.wait()
                w2_.wait()
                w3.wait()

    return k(nf, vt0, vt1, vt2, neigh)


def _sc_scatter(central, pa, p0, p1, p2, zeros_nd):
    E, D = pa.shape
    N = zeros_nd.shape[0]
    BLK = 128
    nblk = E // BLK
    NS = 16
    per = (nblk + NS - 1) // NS
    tiles = N // 8
    rows8 = (tiles // NS) * 8
    n_hi = tiles - NS * (tiles // NS)
    assert n_hi * (rows8 + 8) + (NS - n_hi) * rows8 == N
    mesh = plsc.VectorSubcoreMesh(core_axis_name="c", subcore_axis_name="s")

    @functools.partial(
        pl.kernel,
        out_type=(jax.ShapeDtypeStruct((N, D), jnp.float32),
                  jax.ShapeDtypeStruct((N, D), jnp.float32),
                  jax.ShapeDtypeStruct((N, D), jnp.float32),
                  jax.ShapeDtypeStruct((N, D), jnp.float32)),
        mesh=mesh,
        scratch_types=[
            pltpu.VMEM((1, BLK), jnp.int32),
            pltpu.VMEM((BLK, D), jnp.float32),
            pltpu.VMEM_SHARED((N, D), jnp.float32),
        ],
    )
    def k(central_hbm, pa_hbm, p0_hbm, p1_hbm, p2_hbm, z_hbm,
          ds_hbm, v0_hbm, v1_hbm, v2_hbm,
          idx_v, pay_v, acc):
        cid = lax.axis_index("c")
        sid = lax.axis_index("s")

        def chunk(pay_hbm, out_hbm):
            @pl.when(sid == 0)
            def _():
                pltpu.sync_copy(z_hbm, acc)

            plsc.subcore_barrier()

            @pl.loop(0, per)
            def _(i):
                blk = sid + i * NS

                @pl.when(blk < nblk)
                def _():
                    base = blk * BLK
                    pltpu.sync_copy(central_hbm.at[pl.ds(base, BLK)],
                                    idx_v.at[0])
                    pltpu.sync_copy(pay_hbm.at[pl.ds(base, BLK)], pay_v)
                    pltpu.sync_copy(pay_v, acc.at[idx_v.at[0]], add=True)

            plsc.subcore_barrier()
            @pl.when(sid < n_hi)
            def _():
                start = sid * (rows8 + 8)
                pltpu.sync_copy(acc.at[pl.ds(start, rows8 + 8)],
                                out_hbm.at[pl.ds(start, rows8 + 8)])

            @pl.when(sid >= n_hi)
            def _():
                start = n_hi * (rows8 + 8) + (sid - n_hi) * rows8
                pltpu.sync_copy(acc.at[pl.ds(start, rows8)],
                                out_hbm.at[pl.ds(start, rows8)])

            plsc.subcore_barrier()

        @pl.when(cid == 0)
        def _():
            chunk(pa_hbm, ds_hbm)
            chunk(p0_hbm, v0_hbm)

        @pl.when(cid == 1)
        def _():
            chunk(p1_hbm, v1_hbm)
            chunk(p2_hbm, v2_hbm)

    return k(central, pa, p0, p1, p2, zeros_nd)


def kernel(vector_embeddings, scalar_embeddings, edge_vectors, edge_index,
           Wf, bf, w1, b1, w2, b2):
    N, D = scalar_embeddings.shape
    TD = 3 * D
    central = edge_index[0].astype(jnp.int32)
    neigh = edge_index[1].astype(jnp.int32)

    nf = _mlp(scalar_embeddings, w1, b1, w2, b2)
    vt0 = vector_embeddings[:, :, 0]
    vt1 = vector_embeddings[:, :, 1]
    vt2 = vector_embeddings[:, :, 2]

    gnf, gv0, gv1, gv2 = _sc_gather(nf, vt0, vt1, vt2, neigh)

    wf_pad = jnp.zeros((128, TD), jnp.float32).at[:NBASIS].set(Wf)
    pa, p0, p1, p2 = _payloads(edge_vectors, gnf, gv0, gv1, gv2, wf_pad, bf)

    zeros_nd = jnp.zeros((N, D), jnp.float32)
    ds, v0, v1, v2 = _sc_scatter(central, pa, p0, p1, p2, zeros_nd)

    delta_v = jnp.stack([v0, v1, v2], axis=-1)
    return (delta_v, ds)


# speedup vs baseline: 16.7883x; 1.2986x over previous
import functools

import jax
import jax.numpy as jnp
from jax import lax
from jax.experimental import pallas as pl
from jax.experimental.pallas import tpu as pltpu
from jax.experimental.pallas import tpu_sc as plsc

CUTOFF = 5.0
NBASIS = 20


def _mlp_body(s_ref, w1_ref, b1_ref, w2_ref, b2_ref, o_ref):
    x = s_ref[...]
    h = jnp.dot(x, w1_ref[...], preferred_element_type=jnp.float32,
                precision=lax.Precision.HIGHEST) + b1_ref[...]
    h = h * jax.nn.sigmoid(h)
    o_ref[...] = jnp.dot(h, w2_ref[...], preferred_element_type=jnp.float32,
                         precision=lax.Precision.HIGHEST) + b2_ref[...]


def _lo16(x):
    return lax.bitcast_convert_type(lax.shift_left(x, 16), jnp.float32)


def _hi16(x):
    return lax.bitcast_convert_type(
        lax.bitwise_and(x, jnp.int32(-65536)), jnp.float32)


def _payload_body(ev_ref, g_ref, wf_ref, bf_ref,
                  pa_ref, p0_ref, p1_ref, p2_ref):
    ev = ev_ref[...]
    e0 = ev[:, 0:1]
    e1 = ev[:, 1:2]
    e2 = ev[:, 2:3]
    d = jnp.sqrt(e0 * e0 + e1 * e1 + e2 * e2)
    inv_d = 1.0 / d

    B = ev.shape[0]
    n = lax.broadcasted_iota(jnp.int32, (B, 128), 1).astype(jnp.float32) + 1.0
    bes = jnp.sqrt(2.0 / CUTOFF) * jnp.sin(n * (jnp.pi / CUTOFF) * d) * inv_d
    filt = jnp.dot(bes, wf_ref[...], preferred_element_type=jnp.float32,
                   precision=lax.Precision.HIGHEST) + bf_ref[...]

    u = d * (1.0 / CUTOFF)
    u2 = u * u
    u3 = u2 * u
    u6 = u3 * u3
    u7 = u6 * u
    u8 = u7 * u
    env = 1.0 - 28.0 * u6 + 48.0 * u7 - 21.0 * u8
    env = jnp.where(d < CUTOFF, env, 0.0)
    filt = filt * env

    pk = g_ref[...]
    TD = pk.shape[1]
    D = TD // 3
    az = pk[:, :D]
    bc = pk[:, D:2 * D]
    xy = pk[:, 2 * D:]
    a = _lo16(az)
    vz = _hi16(az)
    b = _lo16(bc)
    c = _hi16(bc)
    vx = _lo16(xy)
    vy = _hi16(xy)

    bb = b * filt[:, D:2 * D]
    cc = c * filt[:, 2 * D:]
    pa_ref[...] = a * filt[:, :D]
    p0_ref[...] = bb * (e0 * inv_d) + cc * vx
    p1_ref[...] = bb * (e1 * inv_d) + cc * vy
    p2_ref[...] = bb * (e2 * inv_d) + cc * vz


def _mlp(scalar_embeddings, w1, b1, w2, b2):
    N, D = scalar_embeddings.shape
    TD = w2.shape[1]
    BN = 2000
    grid = (N // BN,)
    return pl.pallas_call(
        _mlp_body,
        grid=grid,
        in_specs=[
            pl.BlockSpec((BN, D), lambda i: (i, 0)),
            pl.BlockSpec((D, D), lambda i: (0, 0)),
            pl.BlockSpec((1, D), lambda i: (0, 0)),
            pl.BlockSpec((D, TD), lambda i: (0, 0)),
            pl.BlockSpec((1, TD), lambda i: (0, 0)),
        ],
        out_specs=pl.BlockSpec((BN, TD), lambda i: (i, 0)),
        out_shape=jax.ShapeDtypeStruct((N, TD), jnp.float32),
    )(scalar_embeddings, w1, b1.reshape(1, D), w2, b2.reshape(1, TD))


def _payloads(edge_vectors, g_all, wf_pad, bf):
    E, TD = g_all.shape
    D = TD // 3
    BE = 1600
    grid = (E // BE,)
    return pl.pallas_call(
        _payload_body,
        grid=grid,
        in_specs=[
            pl.BlockSpec((BE, 3), lambda i: (i, 0)),
            pl.BlockSpec((BE, TD), lambda i: (i, 0)),
            pl.BlockSpec((128, TD), lambda i: (0, 0)),
            pl.BlockSpec((1, TD), lambda i: (0, 0)),
        ],
        out_specs=[pl.BlockSpec((BE, D), lambda i: (i, 0))] * 4,
        out_shape=[jax.ShapeDtypeStruct((E, D), jnp.float32)] * 4,
    )(edge_vectors, g_all, wf_pad, bf.reshape(1, TD))


BLK = 128


def _sc_gather(t_all, neigh_pad):
    N, TD = t_all.shape
    E = neigh_pad.shape[0] - BLK
    nblk = E // BLK
    NW = 32
    base_cnt = nblk // NW
    n_extra = nblk - NW * base_cnt
    cnt_max = base_cnt + (1 if n_extra else 0)
    pairs = (cnt_max + 1) // 2
    mesh = plsc.VectorSubcoreMesh(core_axis_name="c", subcore_axis_name="s")

    @functools.partial(
        pl.kernel,
        out_type=jax.ShapeDtypeStruct((E, TD), jnp.int32),
        mesh=mesh,
        scratch_types=[
            pltpu.VMEM((cnt_max * BLK,), jnp.int32),
            pltpu.VMEM((BLK, TD), jnp.int32),
            pltpu.VMEM((BLK, TD), jnp.int32),
            pltpu.SemaphoreType.DMA,
            pltpu.SemaphoreType.DMA,
            pltpu.SemaphoreType.DMA,
            pltpu.SemaphoreType.DMA,
        ],
    )
    def k(t_hbm, idx_hbm, g_hbm, idx_all, buf0, buf1, gs0, gs1, ws0, ws1):
        wid = lax.axis_index("s") * 2 + lax.axis_index("c")
        base_blk = wid * base_cnt + jnp.minimum(wid, n_extra)
        cnt = jnp.where(wid < n_extra, base_cnt + 1, base_cnt)
        ebase = base_blk * BLK

        pltpu.sync_copy(idx_hbm.at[pl.ds(ebase, cnt_max * BLK)], idx_all)

        bufs = (buf0, buf1)
        gsems = (gs0, gs1)
        wsems = (ws0, ws1)

        def gissue(i, b):
            pltpu.async_copy(
                t_hbm.at[idx_all.at[pl.ds(i * BLK, BLK)]], bufs[b], gsems[b])

        def gwait(b):
            pltpu.make_async_copy(
                t_hbm.at[pl.ds(0, BLK)], bufs[b], gsems[b]).wait()

        def wissue(i, b):
            pltpu.async_copy(
                bufs[b], g_hbm.at[pl.ds(ebase + i * BLK, BLK)], wsems[b])

        def wwait(b):
            pltpu.make_async_copy(
                bufs[b], g_hbm.at[pl.ds(0, BLK)], wsems[b]).wait()

        gissue(0, 0)

        @pl.loop(0, pairs)
        def _(ii):
            for b in range(2):
                i = ii * 2 + b

                @pl.when(i < cnt)
                def _(i=i, b=b):
                    gwait(b)
                    wissue(i, b)

                    @pl.when(i + 1 < cnt)
                    def _():
                        @pl.when(i >= 1)
                        def _():
                            wwait(1 - b)

                        gissue(i + 1, 1 - b)

        wwait(0)
        wwait(1)

    return k(t_all, neigh_pad)


def _sc_scatter(central2d, pa, p0, p1, p2, zeros_nd):
    E, D = pa.shape
    N = zeros_nd.shape[0]
    nblk = pa.shape[0] // BLK
    NS = 16
    CH = ((nblk + NS - 1) // NS + 7) // 8 * 8
    n_full = nblk // CH
    rem = nblk - n_full * CH
    cnt_max = CH
    NBUF = 2
    rounds = (cnt_max + NBUF - 1) // NBUF
    tiles = N // 8
    rows8 = (tiles // NS) * 8
    n_hi = tiles - NS * (tiles // NS)
    assert n_hi * (rows8 + 8) + (NS - n_hi) * rows8 == N
    mesh = plsc.VectorSubcoreMesh(core_axis_name="c", subcore_axis_name="s")

    @functools.partial(
        pl.kernel,
        out_type=(jax.ShapeDtypeStruct((N, D), jnp.float32),
                  jax.ShapeDtypeStruct((N, D), jnp.float32),
                  jax.ShapeDtypeStruct((N, D), jnp.float32),
                  jax.ShapeDtypeStruct((N, D), jnp.float32)),
        mesh=mesh,
        scratch_types=[
            pltpu.VMEM((cnt_max, BLK), jnp.int32),
            pltpu.VMEM((BLK, D), jnp.float32),
            pltpu.VMEM((BLK, D), jnp.float32),
            pltpu.VMEM_SHARED((N, D), jnp.float32),
            pltpu.SemaphoreType.DMA,
            pltpu.SemaphoreType.DMA,
            pltpu.SemaphoreType.DMA,
            pltpu.SemaphoreType.DMA,
        ],
    )
    def k(c_hbm, pa_hbm, p0_hbm, p1_hbm, p2_hbm, z_hbm,
          ds_hbm, v0_hbm, v1_hbm, v2_hbm,
          idx2, b0, b1,
          acc, ps0, ps1, ss0, ss1):
        cid = lax.axis_index("c")
        sid = lax.axis_index("s")
        sbase = sid * CH
        cnt = jnp.where(sid < n_full, CH, rem)
        bufs = (b0, b1)
        psems = (ps0, ps1)
        ssems = (ss0, ss1)

        pltpu.sync_copy(c_hbm.at[pl.ds(sbase, CH)], idx2)

        def chunk(pay_hbm, out_hbm):
            @pl.when(sid == 0)
            def _():
                pltpu.sync_copy(z_hbm, acc)

            plsc.subcore_barrier()

            def pissue(i, b):
                pltpu.async_copy(
                    pay_hbm.at[pl.ds((sbase + i) * BLK, BLK)],
                    bufs[b], psems[b])

            def pwait(b):
                pltpu.make_async_copy(
                    pay_hbm.at[pl.ds(0, BLK)], bufs[b], psems[b]).wait()

            def sissue(i, b):
                pltpu.async_copy(bufs[b], acc.at[idx2.at[i]], ssems[b],
                                 add=True)

            def swait(b):
                pltpu.make_async_copy(
                    pay_hbm.at[pl.ds(0, BLK)], bufs[b], ssems[b]).wait()

            for b in range(NBUF - 1):
                @pl.when(b < cnt)
                def _(b=b):
                    pissue(b, b)

            @pl.loop(0, rounds)
            def _(ii):
                for b in range(NBUF):
                    i = ii * NBUF + b

                    @pl.when(i < cnt)
                    def _(i=i, b=b):
                        pwait(b)
                        sissue(i, b)
                        j = i + NBUF - 1

                        @pl.when(j < cnt)
                        def _():
                            nb = (b + NBUF - 1) % NBUF

                            @pl.when(j >= NBUF)
                            def _():
                                swait(nb)

                            pissue(j, nb)

            for b in range(NBUF):
                swait(b)

            plsc.subcore_barrier()

            @pl.when(sid < n_hi)
            def _():
                start = sid * (rows8 + 8)
                pltpu.sync_copy(acc.at[pl.ds(start, rows8 + 8)],
                                out_hbm.at[pl.ds(start, rows8 + 8)])

            @pl.when(sid >= n_hi)
            def _():
                start = n_hi * (rows8 + 8) + (sid - n_hi) * rows8
                pltpu.sync_copy(acc.at[pl.ds(start, rows8)],
                                out_hbm.at[pl.ds(start, rows8)])

            plsc.subcore_barrier()

        @pl.when(cid == 0)
        def _():
            chunk(pa_hbm, ds_hbm)
            chunk(p0_hbm, v0_hbm)

        @pl.when(cid == 1)
        def _():
            chunk(p1_hbm, v1_hbm)
            chunk(p2_hbm, v2_hbm)

    return k(central2d, pa, p0, p1, p2, zeros_nd)


def _pack2(x, y):
    s = jnp.stack([x.astype(jnp.bfloat16), y.astype(jnp.bfloat16)], axis=-1)
    return lax.bitcast_convert_type(s, jnp.int32)


def kernel(vector_embeddings, scalar_embeddings, edge_vectors, edge_index,
           Wf, bf, w1, b1, w2, b2):
    N, D = scalar_embeddings.shape
    TD = 3 * D
    E = edge_vectors.shape[0]
    central = edge_index[0].astype(jnp.int32)
    neigh = edge_index[1].astype(jnp.int32)

    nf = _mlp(scalar_embeddings, w1, b1, w2, b2)
    t_all = jnp.concatenate([
        _pack2(nf[:, :D], vector_embeddings[:, :, 2]),
        _pack2(nf[:, D:2 * D], nf[:, 2 * D:]),
        _pack2(vector_embeddings[:, :, 0], vector_embeddings[:, :, 1]),
    ], axis=1)

    neigh_pad = jnp.concatenate([neigh, jnp.zeros((BLK,), jnp.int32)])
    g_all = _sc_gather(t_all, neigh_pad)

    wf_pad = jnp.zeros((128, TD), jnp.float32).at[:NBASIS].set(Wf)
    pa, p0, p1, p2 = _payloads(edge_vectors, g_all, wf_pad, bf)

    zeros_nd = jnp.zeros((N, D), jnp.float32)
    nblk = E // BLK
    ch = ((nblk + 15) // 16 + 7) // 8 * 8
    central2d = jnp.concatenate([
        central.reshape(nblk, BLK),
        jnp.zeros((16 * ch - nblk, BLK), jnp.int32)])
    ds, v0, v1, v2 = _sc_scatter(central2d, pa, p0, p1, p2, zeros_nd)

    delta_v = jnp.stack([v0, v1, v2], axis=-1)
    return (delta_v, ds)
